# Initial kernel scaffold; baseline (speedup 1.0000x reference)
import jax
import jax.numpy as jnp
from jax.experimental import pallas as pl


def kernel(z, x, senders, receivers, edge_weight, edge_attr, W1, b1, embed, W2, b2):
    raise NotImplementedError("write your pallas kernel here")


# speedup vs baseline: 1.6911x; 1.6911x over previous
import functools
import math

import jax
import jax.numpy as jnp
from jax import lax
from jax.experimental import pallas as pl
from jax.experimental.pallas import tpu as pltpu
from jax.experimental.pallas import tpu_sc as plsc

N = 10000
E = 320000
H = 128
D_EDGE = 16
MAX_SPECIES = 100
CUTOFF_UPPER = 5.0

L = 16
NW = 32
CH = 128
NCH = E // CH
ROWS_PER_TILE = 624

BE = 2000
BN = 2000


def _edge_feat_body(ea_ref, ew_ref, w1_ref, b1_ref, out_ref):
    ew = ew_ref[...]
    c = 0.5 * (jnp.cos(ew * (math.pi / CUTOFF_UPPER)) + 1.0)
    c = c * (ew < CUTOFF_UPPER).astype(jnp.float32)
    w = jnp.dot(ea_ref[...], w1_ref[...], preferred_element_type=jnp.float32)
    out_ref[...] = (w + b1_ref[...]) * c


def _edge_features(edge_attr, edge_weight, W1, b1):
    ew2 = edge_weight.reshape(E, 1)
    return pl.pallas_call(
        _edge_feat_body,
        grid=(E // BE,),
        in_specs=[
            pl.BlockSpec((BE, D_EDGE), lambda i: (i, 0)),
            pl.BlockSpec((BE, 1), lambda i: (i, 0)),
            pl.BlockSpec((D_EDGE, H), lambda i: (0, 0)),
            pl.BlockSpec((1, H), lambda i: (0, 0)),
        ],
        out_specs=pl.BlockSpec((BE, H), lambda i: (i, 0)),
        out_shape=jax.ShapeDtypeStruct((E, H), jnp.float32),
    )(edge_attr, ew2, W1, b1.reshape(1, H))


def _embed_body(z_ref, emb_ref, out_ref):
    zb = z_ref[...]
    iota = lax.broadcasted_iota(jnp.int32, (BN, MAX_SPECIES), 1)
    oh = (zb == iota).astype(jnp.float32)
    out_ref[...] = jnp.dot(oh, emb_ref[...], preferred_element_type=jnp.float32)


def _embed_lookup(z, embed):
    z2 = z.astype(jnp.int32).reshape(N, 1)
    return pl.pallas_call(
        _embed_body,
        grid=(N // BN,),
        in_specs=[
            pl.BlockSpec((BN, 1), lambda i: (i, 0)),
            pl.BlockSpec((MAX_SPECIES, H), lambda i: (0, 0)),
        ],
        out_specs=pl.BlockSpec((BN, H), lambda i: (i, 0)),
        out_shape=jax.ShapeDtypeStruct((N, H), jnp.float32),
    )(z2, embed)


def _sc_scatter_body(wc_hbm, xn_hbm, snd_hbm, rcv_hbm, out_hbm,
                     sidx, ridx, wcv, xsv, aggsh, sem1, sem2):
    c = lax.axis_index("c")
    s = lax.axis_index("s")
    wid = s * 2 + c

    zv = jnp.zeros((L,), jnp.float32)

    def _zero_row(i, carry):
        for cc in range(H // L):
            wcv[i, pl.ds(cc * L, L)] = zv
        return carry

    lax.fori_loop(0, CH, _zero_row, 0)
    row0 = s * ROWS_PER_TILE
    for k in range(ROWS_PER_TILE // CH):
        pltpu.sync_copy(wcv, aggsh.at[pl.ds(row0 + k * CH, CH)])
    rem = ROWS_PER_TILE % CH
    if rem:
        pltpu.sync_copy(wcv.at[pl.ds(0, rem)],
                        aggsh.at[pl.ds(row0 + ROWS_PER_TILE - rem, rem)])

    @pl.when(s == 15)
    def _zero_tail():
        pltpu.sync_copy(wcv.at[pl.ds(0, N - 16 * ROWS_PER_TILE)],
                        aggsh.at[pl.ds(16 * ROWS_PER_TILE,
                                       N - 16 * ROWS_PER_TILE)])

    plsc.subcore_barrier()

    n_iters = 78 + jnp.where(wid < NCH - 78 * NW, 1, 0)

    def _chunk(j, carry):
        cid = j * NW + wid
        base = cid * CH
        pltpu.sync_copy(snd_hbm.at[pl.ds(base, CH)], sidx)
        pltpu.sync_copy(rcv_hbm.at[pl.ds(base, CH)], ridx)
        cp1 = pltpu.async_copy(xn_hbm.at[sidx], xsv, sem1)
        cp2 = pltpu.async_copy(wc_hbm.at[pl.ds(base, CH)], wcv, sem2)
        cp1.wait()
        cp2.wait()

        def _mul_row(i, cy):
            for cc in range(H // L):
                sl = pl.ds(cc * L, L)
                xsv[i, sl] = xsv[i, sl] * wcv[i, sl]
            return cy

        lax.fori_loop(0, CH, _mul_row, 0)
        pltpu.sync_copy(xsv, aggsh.at[ridx], add=True)
        return carry

    lax.fori_loop(0, n_iters, _chunk, 0)
    plsc.subcore_barrier()

    pltpu.sync_copy(aggsh.at[pl.ds(row0, ROWS_PER_TILE)],
                    out_hbm.at[c, pl.ds(row0, ROWS_PER_TILE)])

    @pl.when(s == 15)
    def _write_tail():
        pltpu.sync_copy(aggsh.at[pl.ds(16 * ROWS_PER_TILE,
                                       N - 16 * ROWS_PER_TILE)],
                        out_hbm.at[c, pl.ds(16 * ROWS_PER_TILE,
                                            N - 16 * ROWS_PER_TILE)])


def _sc_scatter(wc, xn, senders, receivers):
    mesh = plsc.VectorSubcoreMesh(core_axis_name="c", subcore_axis_name="s")
    f = pl.kernel(
        _sc_scatter_body,
        out_type=jax.ShapeDtypeStruct((2, N, H), jnp.float32),
        mesh=mesh,
        scratch_types=[
            pltpu.VMEM((CH,), jnp.int32),
            pltpu.VMEM((CH,), jnp.int32),
            pltpu.VMEM((CH, H), jnp.float32),
            pltpu.VMEM((CH, H), jnp.float32),
            pltpu.VMEM_SHARED((N, H), jnp.float32),
            pltpu.SemaphoreType.DMA,
            pltpu.SemaphoreType.DMA,
        ],
    )
    return f(wc, xn, senders, receivers)


def _out_body(x_ref, p0_ref, p1_ref, w2a_ref, w2b_ref, b2_ref, o_ref):
    agg = p0_ref[...] + p1_ref[...]
    o_ref[...] = (
        jnp.dot(x_ref[...], w2a_ref[...], preferred_element_type=jnp.float32)
        + jnp.dot(agg, w2b_ref[...], preferred_element_type=jnp.float32)
        + b2_ref[...]
    )


def _out_proj(x, p0, p1, W2, b2):
    return pl.pallas_call(
        _out_body,
        grid=(N // BN,),
        in_specs=[
            pl.BlockSpec((BN, H), lambda i: (i, 0)),
            pl.BlockSpec((BN, H), lambda i: (i, 0)),
            pl.BlockSpec((BN, H), lambda i: (i, 0)),
            pl.BlockSpec((H, H), lambda i: (0, 0)),
            pl.BlockSpec((H, H), lambda i: (0, 0)),
            pl.BlockSpec((1, H), lambda i: (0, 0)),
        ],
        out_specs=pl.BlockSpec((BN, H), lambda i: (i, 0)),
        out_shape=jax.ShapeDtypeStruct((N, H), jnp.float32),
    )(x, p0, p1, W2[:H], W2[H:], b2.reshape(1, H))


def kernel(z, x, senders, receivers, edge_weight, edge_attr, W1, b1, embed, W2, b2):
    senders = senders.astype(jnp.int32)
    receivers = receivers.astype(jnp.int32)
    wc = _edge_features(edge_attr, edge_weight, W1, b1)
    xn = _embed_lookup(z, embed)
    partial = _sc_scatter(wc, xn, senders, receivers)
    return _out_proj(x, partial[0], partial[1], W2, b2)


# speedup vs baseline: 2.8391x; 1.6788x over previous
import functools
import math

import numpy as np
import jax
import jax.numpy as jnp
from jax import lax
from jax.experimental import pallas as pl
from jax.experimental.pallas import tpu as pltpu
from jax.experimental.pallas import tpu_sc as plsc

N = 10000
E = 320000
H = 128
D_EDGE = 16
MAX_SPECIES = 100
CUTOFF_UPPER = 5.0

L = 16
NW = 32
CH = 40
JPT = E // (NW * CH)
ROWS_PER_TILE = 624

BE = 4000
BN = 2000


def _cutoff_body(ew_ref, out_ref):
    ew = ew_ref[...]
    c = 0.5 * (jnp.cos(ew * (math.pi / CUTOFF_UPPER)) + 1.0)
    out_ref[...] = c * (ew < CUTOFF_UPPER).astype(jnp.float32)


def _cutoff(edge_weight):
    ew2 = edge_weight.reshape(E // H, H)
    out = pl.pallas_call(
        _cutoff_body,
        out_shape=jax.ShapeDtypeStruct((E // H, H), jnp.float32),
    )(ew2)
    return out.reshape(E)


def _edge_matmul_body(ea_ref, w1_ref, b1_ref, out_ref):
    w = jnp.dot(ea_ref[...], w1_ref[...], preferred_element_type=jnp.float32)
    out_ref[...] = w + b1_ref[...]


def _edge_features(edge_attr, W1, b1):
    return pl.pallas_call(
        _edge_matmul_body,
        grid=(E // BE,),
        in_specs=[
            pl.BlockSpec((BE, D_EDGE), lambda i: (i, 0)),
            pl.BlockSpec((D_EDGE, H), lambda i: (0, 0)),
            pl.BlockSpec((1, H), lambda i: (0, 0)),
        ],
        out_specs=pl.BlockSpec((BE, H), lambda i: (i, 0)),
        out_shape=jax.ShapeDtypeStruct((E, H), jnp.float32),
    )(edge_attr, W1, b1.reshape(1, H))


def _embed_body(z_ref, emb_ref, out_ref):
    zb = z_ref[...]
    iota = lax.broadcasted_iota(jnp.int32, (BN, MAX_SPECIES), 1)
    oh = (zb == iota).astype(jnp.float32)
    out_ref[...] = jnp.dot(oh, emb_ref[...], preferred_element_type=jnp.float32)


def _embed_lookup(z, embed):
    z2 = z.astype(jnp.int32).reshape(N, 1)
    return pl.pallas_call(
        _embed_body,
        grid=(N // BN,),
        in_specs=[
            pl.BlockSpec((BN, 1), lambda i: (i, 0)),
            pl.BlockSpec((MAX_SPECIES, H), lambda i: (0, 0)),
        ],
        out_specs=pl.BlockSpec((BN, H), lambda i: (i, 0)),
        out_shape=jax.ShapeDtypeStruct((N, H), jnp.float32),
    )(z2, embed)


def _sc_scatter_body(wc_hbm, xn_hbm, cw_hbm, snd_hbm, rcv_hbm, out_hbm,
                     *sc):
    sidx = sc[0:3]
    ridx = sc[3:6]
    cwv = sc[6:9]
    wcv = sc[9:12]
    xsv = sc[12:15]
    aggsh = sc[15]
    sem_cp = sc[16:19]
    sem_g = sc[19:22]
    sem_sc = sc[22:25]

    c = lax.axis_index("c")
    s = lax.axis_index("s")
    wid = s * 2 + c
    tile_base = wid * (JPT * CH)

    zv = jnp.zeros((L,), jnp.float32)

    def _zero_row(i, carry):
        for cc in range(H // L):
            xsv[0][i, pl.ds(cc * L, L)] = zv
        return carry

    lax.fori_loop(0, CH, _zero_row, 0)
    row0 = s * ROWS_PER_TILE
    for k in range(ROWS_PER_TILE // CH):
        pltpu.sync_copy(xsv[0], aggsh.at[pl.ds(row0 + k * CH, CH)])
    rem = ROWS_PER_TILE % CH
    if rem:
        pltpu.sync_copy(xsv[0].at[pl.ds(0, rem)],
                        aggsh.at[pl.ds(row0 + ROWS_PER_TILE - rem, rem)])

    @pl.when(s == 15)
    def _zero_tail():
        pltpu.sync_copy(xsv[0].at[pl.ds(0, N - 16 * ROWS_PER_TILE)],
                        aggsh.at[pl.ds(16 * ROWS_PER_TILE,
                                       N - 16 * ROWS_PER_TILE)])

    plsc.subcore_barrier()

    def s1(j, X):
        base = tile_base + j * CH
        pltpu.async_copy(snd_hbm.at[pl.ds(base, CH)], sidx[X], sem_cp[X])
        pltpu.async_copy(cw_hbm.at[pl.ds(base, CH)],
                         cwv[X].at[pl.ds(0, CH)], sem_cp[X])
        pltpu.async_copy(wc_hbm.at[pl.ds(base, CH)], wcv[X], sem_cp[X])

    def s2(j, X, first):
        if not first:
            pltpu.make_async_copy(xsv[X], aggsh.at[ridx[X]], sem_sc[X]).wait()
        pltpu.make_async_copy(snd_hbm.at[pl.ds(0, CH)], sidx[X],
                              sem_cp[X]).wait()
        pltpu.make_async_copy(cw_hbm.at[pl.ds(0, CH)],
                              cwv[X].at[pl.ds(0, CH)], sem_cp[X]).wait()
        pltpu.make_async_copy(wc_hbm.at[pl.ds(0, CH)], wcv[X],
                              sem_cp[X]).wait()
        base = tile_base + j * CH
        pltpu.async_copy(rcv_hbm.at[pl.ds(base, CH)], ridx[X], sem_g[X])
        pltpu.async_copy(xn_hbm.at[sidx[X]], xsv[X], sem_g[X])

    def _mul(X):
        def _rows(q0, nrows):
            cw16 = cwv[X][pl.ds(q0, L)]
            for r in range(nrows):
                i = q0 + r
                cs = cw16.at[jnp.zeros((L,), jnp.int32) + r].get(
                    mode="promise_in_bounds")
                for g in range(H // L):
                    sl = pl.ds(g * L, L)
                    xsv[X][i, sl] = xsv[X][i, sl] * (wcv[X][i, sl] * cs)

        def _mul_grp(q, cy):
            _rows(q * L, L)
            return cy

        lax.fori_loop(0, CH // L, _mul_grp, 0)
        if CH % L:
            _rows(CH - CH % L, CH % L)

    def s3(j, X, last):
        pltpu.make_async_copy(rcv_hbm.at[pl.ds(0, CH)], ridx[X],
                              sem_g[X]).wait()
        pltpu.make_async_copy(xn_hbm.at[sidx[X]], xsv[X], sem_g[X]).wait()
        _mul(X)
        if last:
            pltpu.sync_copy(xsv[X], aggsh.at[ridx[X]], add=True)
        else:
            pltpu.async_copy(xsv[X], aggsh.at[ridx[X]], sem_sc[X], add=True)

    for t in range(3):
        s1(t, t)
    for t in range(3):
        s2(t, t, True)
        s3(t, t, False)
        s1(t + 3, t)

    def _steady(k, carry):
        for t in range(3):
            j = 3 * k + t
            s2(j, t, False)
            s3(j, t, False)
            s1(j + 3, t)
        return carry

    assert JPT == 250
    lax.fori_loop(1, 82, _steady, 0)

    jt = 246
    s2(jt, 0, False)
    s3(jt, 0, False)
    s1(jt + 3, 0)
    s2(jt + 1, 1, False)
    s3(jt + 1, 1, True)
    s2(jt + 2, 2, False)
    s3(jt + 2, 2, True)
    s2(jt + 3, 0, False)
    s3(jt + 3, 0, True)

    plsc.subcore_barrier()

    pltpu.sync_copy(aggsh.at[pl.ds(row0, ROWS_PER_TILE)],
                    out_hbm.at[c, pl.ds(row0, ROWS_PER_TILE)])

    @pl.when(s == 15)
    def _write_tail():
        pltpu.sync_copy(aggsh.at[pl.ds(16 * ROWS_PER_TILE,
                                       N - 16 * ROWS_PER_TILE)],
                        out_hbm.at[c, pl.ds(16 * ROWS_PER_TILE,
                                            N - 16 * ROWS_PER_TILE)])


def _sc_scatter(wc, xn, cw, senders, receivers):
    mesh = plsc.VectorSubcoreMesh(core_axis_name="c", subcore_axis_name="s")
    f = pl.kernel(
        _sc_scatter_body,
        out_type=jax.ShapeDtypeStruct((2, N, H), jnp.float32),
        mesh=mesh,
        compiler_params=pltpu.CompilerParams(needs_layout_passes=False),
        scratch_types=(
            [pltpu.VMEM((CH,), jnp.int32) for _ in range(3)]
            + [pltpu.VMEM((CH,), jnp.int32) for _ in range(3)]
            + [pltpu.VMEM((CH + L,), jnp.float32) for _ in range(3)]
            + [pltpu.VMEM((CH, H), jnp.float32) for _ in range(3)]
            + [pltpu.VMEM((CH, H), jnp.float32) for _ in range(3)]
            + [pltpu.VMEM_SHARED((N, H), jnp.float32)]
            + [pltpu.SemaphoreType.DMA for _ in range(9)]
        ),
    )
    return f(wc, xn, cw, senders, receivers)


def _out_body(x_ref, p0_ref, p1_ref, w2a_ref, w2b_ref, b2_ref, o_ref):
    agg = p0_ref[...] + p1_ref[...]
    o_ref[...] = (
        jnp.dot(x_ref[...], w2a_ref[...], preferred_element_type=jnp.float32)
        + jnp.dot(agg, w2b_ref[...], preferred_element_type=jnp.float32)
        + b2_ref[...]
    )


def _out_proj(x, p0, p1, W2, b2):
    return pl.pallas_call(
        _out_body,
        grid=(N // BN,),
        in_specs=[
            pl.BlockSpec((BN, H), lambda i: (i, 0)),
            pl.BlockSpec((BN, H), lambda i: (i, 0)),
            pl.BlockSpec((BN, H), lambda i: (i, 0)),
            pl.BlockSpec((H, H), lambda i: (0, 0)),
            pl.BlockSpec((H, H), lambda i: (0, 0)),
            pl.BlockSpec((1, H), lambda i: (0, 0)),
        ],
        out_specs=pl.BlockSpec((BN, H), lambda i: (i, 0)),
        out_shape=jax.ShapeDtypeStruct((N, H), jnp.float32),
    )(x, p0, p1, W2[:H], W2[H:], b2.reshape(1, H))


def kernel(z, x, senders, receivers, edge_weight, edge_attr, W1, b1, embed, W2, b2):
    senders = senders.astype(jnp.int32)
    receivers = receivers.astype(jnp.int32)
    cw = _cutoff(edge_weight)
    wc = _edge_features(edge_attr, W1, b1)
    xn = _embed_lookup(z, embed)
    partial = _sc_scatter(wc, xn, cw, senders, receivers)
    return _out_proj(x, partial[0], partial[1], W2, b2)


# speedup vs baseline: 3.3113x; 1.1663x over previous
import functools
import math

import numpy as np
import jax
import jax.numpy as jnp
from jax import lax
from jax.experimental import pallas as pl
from jax.experimental.pallas import tpu as pltpu
from jax.experimental.pallas import tpu_sc as plsc

N = 10000
E = 320000
H = 128
D_EDGE = 16
MAX_SPECIES = 100
CUTOFF_UPPER = 5.0

L = 16
NW = 32
CH = 40
JPT = E // (NW * CH)
ROWS_PER_TILE = 624

BE = 3200
BN = 2000


def _cutoff_body(ew_ref, out_ref):
    ew = ew_ref[...]
    c = 0.5 * (jnp.cos(ew * (math.pi / CUTOFF_UPPER)) + 1.0)
    out_ref[...] = c * (ew < CUTOFF_UPPER).astype(jnp.float32)


def _cutoff(edge_weight):
    ew2 = edge_weight.reshape(E // H, H)
    return pl.pallas_call(
        _cutoff_body,
        out_shape=jax.ShapeDtypeStruct((E // H, H), jnp.float32),
    )(ew2)


def _edge_matmul_body(ea_ref, w1_ref, b1_ref, cw_ref, out_ref):
    w = jnp.dot(ea_ref[...], w1_ref[...], preferred_element_type=jnp.float32)
    w = w + b1_ref[...]
    c = cw_ref[0]
    w3 = w.reshape(BE // H, H, H) * c[:, :, None]
    out_ref[...] = w3.reshape(BE, H)


def _edge_features(edge_attr, W1, b1, cw2):
    return pl.pallas_call(
        _edge_matmul_body,
        grid=(E // BE,),
        in_specs=[
            pl.BlockSpec((BE, D_EDGE), lambda i: (i, 0)),
            pl.BlockSpec((D_EDGE, H), lambda i: (0, 0)),
            pl.BlockSpec((1, H), lambda i: (0, 0)),
            pl.BlockSpec((1, BE // H, H), lambda i: (i, 0, 0)),
        ],
        out_specs=pl.BlockSpec((BE, H), lambda i: (i, 0)),
        out_shape=jax.ShapeDtypeStruct((E, H), jnp.float32),
    )(edge_attr, W1, b1.reshape(1, H),
      cw2.reshape(E // BE, BE // H, H))


def _embed_body(z_ref, emb_ref, out_ref):
    zb = z_ref[...]
    iota = lax.broadcasted_iota(jnp.int32, (BN, MAX_SPECIES), 1)
    oh = (zb == iota).astype(jnp.float32)
    out_ref[...] = jnp.dot(oh, emb_ref[...], preferred_element_type=jnp.float32)


def _embed_lookup(z, embed):
    z2 = z.astype(jnp.int32).reshape(N, 1)
    return pl.pallas_call(
        _embed_body,
        grid=(N // BN,),
        in_specs=[
            pl.BlockSpec((BN, 1), lambda i: (i, 0)),
            pl.BlockSpec((MAX_SPECIES, H), lambda i: (0, 0)),
        ],
        out_specs=pl.BlockSpec((BN, H), lambda i: (i, 0)),
        out_shape=jax.ShapeDtypeStruct((N, H), jnp.float32),
    )(z2, embed)


def _sc_scatter_body(wc_hbm, xn_hbm, snd_hbm, rcv_hbm, out_hbm,
                     *sc):
    sidx = sc[0:3]
    ridx = sc[3:6]
    wcv = sc[6:9]
    xsv = sc[9:12]
    aggsh = sc[12]
    sem_cp = sc[13:16]
    sem_g = sc[16:19]
    sem_sc = sc[19:22]

    c = lax.axis_index("c")
    s = lax.axis_index("s")
    wid = s * 2 + c
    tile_base = wid * (JPT * CH)

    zv = jnp.zeros((L,), jnp.float32)

    def _zero_row(i, carry):
        for cc in range(H // L):
            xsv[0][i, pl.ds(cc * L, L)] = zv
        return carry

    lax.fori_loop(0, CH, _zero_row, 0)
    row0 = s * ROWS_PER_TILE
    for k in range(ROWS_PER_TILE // CH):
        pltpu.sync_copy(xsv[0], aggsh.at[pl.ds(row0 + k * CH, CH)])
    rem = ROWS_PER_TILE % CH
    if rem:
        pltpu.sync_copy(xsv[0].at[pl.ds(0, rem)],
                        aggsh.at[pl.ds(row0 + ROWS_PER_TILE - rem, rem)])

    @pl.when(s == 15)
    def _zero_tail():
        pltpu.sync_copy(xsv[0].at[pl.ds(0, N - 16 * ROWS_PER_TILE)],
                        aggsh.at[pl.ds(16 * ROWS_PER_TILE,
                                       N - 16 * ROWS_PER_TILE)])

    plsc.subcore_barrier()

    def s1(j, X):
        base = tile_base + j * CH
        pltpu.async_copy(snd_hbm.at[pl.ds(base, CH)], sidx[X], sem_cp[X])
        pltpu.async_copy(wc_hbm.at[pl.ds(base, CH)], wcv[X], sem_cp[X])

    def s2(j, X, first):
        if not first:
            pltpu.make_async_copy(xsv[X], aggsh.at[ridx[X]], sem_sc[X]).wait()
        pltpu.make_async_copy(snd_hbm.at[pl.ds(0, CH)], sidx[X],
                              sem_cp[X]).wait()
        pltpu.make_async_copy(wc_hbm.at[pl.ds(0, CH)], wcv[X],
                              sem_cp[X]).wait()
        base = tile_base + j * CH
        pltpu.async_copy(rcv_hbm.at[pl.ds(base, CH)], ridx[X], sem_g[X])
        pltpu.async_copy(xn_hbm.at[sidx[X]], xsv[X], sem_g[X])

    def _mul(X):
        def _mul_row(i, cy):
            for g in range(H // L):
                sl = pl.ds(g * L, L)
                xsv[X][i, sl] = xsv[X][i, sl] * wcv[X][i, sl]
            return cy

        lax.fori_loop(0, CH, _mul_row, 0)

    def s3(j, X, last):
        pltpu.make_async_copy(rcv_hbm.at[pl.ds(0, CH)], ridx[X],
                              sem_g[X]).wait()
        pltpu.make_async_copy(xn_hbm.at[sidx[X]], xsv[X], sem_g[X]).wait()
        _mul(X)
        if last:
            pltpu.sync_copy(xsv[X], aggsh.at[ridx[X]], add=True)
        else:
            pltpu.async_copy(xsv[X], aggsh.at[ridx[X]], sem_sc[X], add=True)

    for t in range(3):
        s1(t, t)
    for t in range(3):
        s2(t, t, True)
        s3(t, t, False)
        s1(t + 3, t)

    def _steady(k, carry):
        for t in range(3):
            j = 3 * k + t
            s2(j, t, False)
            s3(j, t, False)
            s1(j + 3, t)
        return carry

    assert JPT == 250
    lax.fori_loop(1, 82, _steady, 0)

    jt = 246
    s2(jt, 0, False)
    s3(jt, 0, False)
    s1(jt + 3, 0)
    s2(jt + 1, 1, False)
    s3(jt + 1, 1, True)
    s2(jt + 2, 2, False)
    s3(jt + 2, 2, True)
    s2(jt + 3, 0, False)
    s3(jt + 3, 0, True)

    plsc.subcore_barrier()

    pltpu.sync_copy(aggsh.at[pl.ds(row0, ROWS_PER_TILE)],
                    out_hbm.at[c, pl.ds(row0, ROWS_PER_TILE)])

    @pl.when(s == 15)
    def _write_tail():
        pltpu.sync_copy(aggsh.at[pl.ds(16 * ROWS_PER_TILE,
                                       N - 16 * ROWS_PER_TILE)],
                        out_hbm.at[c, pl.ds(16 * ROWS_PER_TILE,
                                            N - 16 * ROWS_PER_TILE)])


def _sc_scatter(wc, xn, senders, receivers):
    mesh = plsc.VectorSubcoreMesh(core_axis_name="c", subcore_axis_name="s")
    f = pl.kernel(
        _sc_scatter_body,
        out_type=jax.ShapeDtypeStruct((2, N, H), jnp.float32),
        mesh=mesh,
        compiler_params=pltpu.CompilerParams(needs_layout_passes=False),
        scratch_types=(
            [pltpu.VMEM((CH,), jnp.int32) for _ in range(3)]
            + [pltpu.VMEM((CH,), jnp.int32) for _ in range(3)]
            + [pltpu.VMEM((CH, H), jnp.float32) for _ in range(3)]
            + [pltpu.VMEM((CH, H), jnp.float32) for _ in range(3)]
            + [pltpu.VMEM_SHARED((N, H), jnp.float32)]
            + [pltpu.SemaphoreType.DMA for _ in range(9)]
        ),
    )
    return f(wc, xn, senders, receivers)


def _out_body(x_ref, p0_ref, p1_ref, w2a_ref, w2b_ref, b2_ref, o_ref):
    agg = p0_ref[...] + p1_ref[...]
    o_ref[...] = (
        jnp.dot(x_ref[...], w2a_ref[...], preferred_element_type=jnp.float32)
        + jnp.dot(agg, w2b_ref[...], preferred_element_type=jnp.float32)
        + b2_ref[...]
    )


def _out_proj(x, p0, p1, W2, b2):
    return pl.pallas_call(
        _out_body,
        grid=(N // BN,),
        in_specs=[
            pl.BlockSpec((BN, H), lambda i: (i, 0)),
            pl.BlockSpec((BN, H), lambda i: (i, 0)),
            pl.BlockSpec((BN, H), lambda i: (i, 0)),
            pl.BlockSpec((H, H), lambda i: (0, 0)),
            pl.BlockSpec((H, H), lambda i: (0, 0)),
            pl.BlockSpec((1, H), lambda i: (0, 0)),
        ],
        out_specs=pl.BlockSpec((BN, H), lambda i: (i, 0)),
        out_shape=jax.ShapeDtypeStruct((N, H), jnp.float32),
    )(x, p0, p1, W2[:H], W2[H:], b2.reshape(1, H))


def kernel(z, x, senders, receivers, edge_weight, edge_attr, W1, b1, embed, W2, b2):
    senders = senders.astype(jnp.int32)
    receivers = receivers.astype(jnp.int32)
    cw2 = _cutoff(edge_weight)
    wc = _edge_features(edge_attr, W1, b1, cw2)
    xn = _embed_lookup(z, embed)
    partial = _sc_scatter(wc, xn, senders, receivers)
    return _out_proj(x, partial[0], partial[1], W2, b2)


# speedup vs baseline: 3.9002x; 1.1779x over previous
import functools
import math

import numpy as np
import jax
import jax.numpy as jnp
from jax import lax
from jax.experimental import pallas as pl
from jax.experimental.pallas import tpu as pltpu
from jax.experimental.pallas import tpu_sc as plsc

N = 10000
E = 320000
H = 128
D_EDGE = 16
MAX_SPECIES = 100
CUTOFF_UPPER = 5.0

L = 16
NW = 32
CH = 40
JPT = E // (NW * CH)
ROWS_PER_TILE = 624

BE = 3200
BN = 2000


def _cutoff_body(ew_ref, out_ref):
    ew = ew_ref[...]
    c = 0.5 * (jnp.cos(ew * (math.pi / CUTOFF_UPPER)) + 1.0)
    out_ref[...] = c * (ew < CUTOFF_UPPER).astype(jnp.float32)


def _cutoff(edge_weight):
    ew2 = edge_weight.reshape(E // H, H)
    return pl.pallas_call(
        _cutoff_body,
        out_shape=jax.ShapeDtypeStruct((E // H, H), jnp.float32),
    )(ew2)


def _edge_matmul_body(eat_ref, w1_ref, b1_ref, cw_ref, out_ref):
    w = lax.dot_general(eat_ref[...], w1_ref[...],
                        (((0,), (0,)), ((), ())),
                        preferred_element_type=jnp.float32)
    w = w + b1_ref[...]
    c = cw_ref[0]
    w3 = w.reshape(BE // H, H, H) * c[:, :, None]
    out_ref[...] = w3.reshape(BE, H)


def _edge_features(edge_attr, W1, b1, cw2):
    return pl.pallas_call(
        _edge_matmul_body,
        grid=(E // BE,),
        in_specs=[
            pl.BlockSpec((D_EDGE, BE), lambda i: (0, i)),
            pl.BlockSpec((D_EDGE, H), lambda i: (0, 0)),
            pl.BlockSpec((1, H), lambda i: (0, 0)),
            pl.BlockSpec((1, BE // H, H), lambda i: (i, 0, 0)),
        ],
        out_specs=pl.BlockSpec((BE, H), lambda i: (i, 0)),
        out_shape=jax.ShapeDtypeStruct((E, H), jnp.float32),
    )(edge_attr.T, W1, b1.reshape(1, H),
      cw2.reshape(E // BE, BE // H, H))


def _embed_body(z_ref, emb_ref, out_ref):
    zb = z_ref[...]
    iota = lax.broadcasted_iota(jnp.int32, (BN, MAX_SPECIES), 1)
    oh = (zb == iota).astype(jnp.float32)
    out_ref[...] = jnp.dot(oh, emb_ref[...], preferred_element_type=jnp.float32)


def _embed_lookup(z, embed):
    z2 = z.astype(jnp.int32).reshape(N, 1)
    return pl.pallas_call(
        _embed_body,
        grid=(N // BN,),
        in_specs=[
            pl.BlockSpec((BN, 1), lambda i: (i, 0)),
            pl.BlockSpec((MAX_SPECIES, H), lambda i: (0, 0)),
        ],
        out_specs=pl.BlockSpec((BN, H), lambda i: (i, 0)),
        out_shape=jax.ShapeDtypeStruct((N, H), jnp.float32),
    )(z2, embed)


NB = 4


def _sc_scatter_body(wc_hbm, xn_hbm, snd_hbm, rcv_hbm, out_hbm,
                     *sc):
    sidx = sc[0:NB]
    ridx = sc[NB:2 * NB]
    wcv = sc[2 * NB:3 * NB]
    xsv = sc[3 * NB:4 * NB]
    aggsh = sc[4 * NB]
    sem_cp = sc[4 * NB + 1:5 * NB + 1]
    sem_g = sc[5 * NB + 1:6 * NB + 1]
    sem_sc = sc[6 * NB + 1:7 * NB + 1]

    c = lax.axis_index("c")
    s = lax.axis_index("s")
    wid = s * 2 + c
    tile_base = wid * (JPT * CH)

    zv = jnp.zeros((L,), jnp.float32)

    def _zero_row(i, carry):
        for cc in range(H // L):
            xsv[0][i, pl.ds(cc * L, L)] = zv
        return carry

    lax.fori_loop(0, CH, _zero_row, 0)
    row0 = s * ROWS_PER_TILE
    for k in range(ROWS_PER_TILE // CH):
        pltpu.sync_copy(xsv[0], aggsh.at[pl.ds(row0 + k * CH, CH)])
    rem = ROWS_PER_TILE % CH
    if rem:
        pltpu.sync_copy(xsv[0].at[pl.ds(0, rem)],
                        aggsh.at[pl.ds(row0 + ROWS_PER_TILE - rem, rem)])

    @pl.when(s == 15)
    def _zero_tail():
        pltpu.sync_copy(xsv[0].at[pl.ds(0, N - 16 * ROWS_PER_TILE)],
                        aggsh.at[pl.ds(16 * ROWS_PER_TILE,
                                       N - 16 * ROWS_PER_TILE)])

    plsc.subcore_barrier()

    def s1(j, X):
        base = tile_base + j * CH
        pltpu.async_copy(snd_hbm.at[pl.ds(base, CH)], sidx[X], sem_cp[X])
        pltpu.async_copy(wc_hbm.at[pl.ds(base, CH)], wcv[X], sem_cp[X])

    def s2(j, X, first):
        if not first:
            pltpu.make_async_copy(xsv[X], aggsh.at[ridx[X]], sem_sc[X]).wait()
        pltpu.make_async_copy(snd_hbm.at[pl.ds(0, CH)], sidx[X],
                              sem_cp[X]).wait()
        pltpu.make_async_copy(wc_hbm.at[pl.ds(0, CH)], wcv[X],
                              sem_cp[X]).wait()
        base = tile_base + j * CH
        pltpu.async_copy(rcv_hbm.at[pl.ds(base, CH)], ridx[X], sem_g[X])
        pltpu.async_copy(xn_hbm.at[sidx[X]], xsv[X], sem_g[X])

    def _mul(X):
        @plsc.parallel_loop(0, CH, 1, unroll=2)
        def _mul_row(i):
            for g in range(H // L):
                sl = pl.ds(g * L, L)
                xsv[X][i, sl] = xsv[X][i, sl] * wcv[X][i, sl]

    def s3(j, X, last):
        pltpu.make_async_copy(rcv_hbm.at[pl.ds(0, CH)], ridx[X],
                              sem_g[X]).wait()
        pltpu.make_async_copy(xn_hbm.at[sidx[X]], xsv[X], sem_g[X]).wait()
        _mul(X)
        if last:
            pltpu.sync_copy(xsv[X], aggsh.at[ridx[X]], add=True)
        else:
            pltpu.async_copy(xsv[X], aggsh.at[ridx[X]], sem_sc[X], add=True)

    for t in range(NB):
        s1(t, t)
    for t in range(NB):
        s2(t, t, True)
        s3(t, t, False)
        s1(t + NB, t)

    def _steady(k, carry):
        for t in range(NB):
            j = NB * k + t
            s2(j, t, False)
            s3(j, t, False)
            s1(j + NB, t)
        return carry

    assert JPT == 250 and NB == 4
    lax.fori_loop(1, 61, _steady, 0)

    s2(244, 0, False)
    s3(244, 0, False)
    s1(248, 0)
    s2(245, 1, False)
    s3(245, 1, False)
    s1(249, 1)
    s2(246, 2, False)
    s3(246, 2, True)
    s2(247, 3, False)
    s3(247, 3, True)
    s2(248, 0, False)
    s3(248, 0, True)
    s2(249, 1, False)
    s3(249, 1, True)

    plsc.subcore_barrier()

    pltpu.sync_copy(aggsh.at[pl.ds(row0, ROWS_PER_TILE)],
                    out_hbm.at[c, pl.ds(row0, ROWS_PER_TILE)])

    @pl.when(s == 15)
    def _write_tail():
        pltpu.sync_copy(aggsh.at[pl.ds(16 * ROWS_PER_TILE,
                                       N - 16 * ROWS_PER_TILE)],
                        out_hbm.at[c, pl.ds(16 * ROWS_PER_TILE,
                                            N - 16 * ROWS_PER_TILE)])


def _sc_scatter(wc, xn, senders, receivers):
    mesh = plsc.VectorSubcoreMesh(core_axis_name="c", subcore_axis_name="s")
    f = pl.kernel(
        _sc_scatter_body,
        out_type=jax.ShapeDtypeStruct((2, N, H), jnp.float32),
        mesh=mesh,
        compiler_params=pltpu.CompilerParams(needs_layout_passes=False),
        scratch_types=(
            [pltpu.VMEM((CH,), jnp.int32) for _ in range(NB)]
            + [pltpu.VMEM((CH,), jnp.int32) for _ in range(NB)]
            + [pltpu.VMEM((CH, H), jnp.float32) for _ in range(NB)]
            + [pltpu.VMEM((CH, H), jnp.float32) for _ in range(NB)]
            + [pltpu.VMEM_SHARED((N, H), jnp.float32)]
            + [pltpu.SemaphoreType.DMA for _ in range(3 * NB)]
        ),
    )
    return f(wc, xn, senders, receivers)


def _out_body(x_ref, p0_ref, p1_ref, w2a_ref, w2b_ref, b2_ref, o_ref):
    agg = p0_ref[...] + p1_ref[...]
    o_ref[...] = (
        jnp.dot(x_ref[...], w2a_ref[...], preferred_element_type=jnp.float32)
        + jnp.dot(agg, w2b_ref[...], preferred_element_type=jnp.float32)
        + b2_ref[...]
    )


def _out_proj(x, p0, p1, W2, b2):
    return pl.pallas_call(
        _out_body,
        grid=(N // BN,),
        in_specs=[
            pl.BlockSpec((BN, H), lambda i: (i, 0)),
            pl.BlockSpec((BN, H), lambda i: (i, 0)),
            pl.BlockSpec((BN, H), lambda i: (i, 0)),
            pl.BlockSpec((H, H), lambda i: (0, 0)),
            pl.BlockSpec((H, H), lambda i: (0, 0)),
            pl.BlockSpec((1, H), lambda i: (0, 0)),
        ],
        out_specs=pl.BlockSpec((BN, H), lambda i: (i, 0)),
        out_shape=jax.ShapeDtypeStruct((N, H), jnp.float32),
    )(x, p0, p1, W2[:H], W2[H:], b2.reshape(1, H))


def kernel(z, x, senders, receivers, edge_weight, edge_attr, W1, b1, embed, W2, b2):
    senders = senders.astype(jnp.int32)
    receivers = receivers.astype(jnp.int32)
    cw2 = _cutoff(edge_weight)
    wc = _edge_features(edge_attr, W1, b1, cw2)
    xn = _embed_lookup(z, embed)
    partial = _sc_scatter(wc, xn, senders, receivers)
    return _out_proj(x, partial[0], partial[1], W2, b2)


# speedup vs baseline: 4.2904x; 1.1000x over previous
import functools
import math

import numpy as np
import jax
import jax.numpy as jnp
from jax import lax
from jax.experimental import pallas as pl
from jax.experimental.pallas import tpu as pltpu
from jax.experimental.pallas import tpu_sc as plsc

N = 10000
E = 320000
H = 128
D_EDGE = 16
MAX_SPECIES = 100
CUTOFF_UPPER = 5.0

L = 16
NW = 32
CH = 40
NH = 2
EH = E // NH
JPT = EH // (NW * CH)
ROWS_PER_TILE = 624

BE = 6400
BN = 2000


def _cutoff_body(ew_ref, out_ref):
    ew = ew_ref[...]
    c = 0.5 * (jnp.cos(ew * (math.pi / CUTOFF_UPPER)) + 1.0)
    out_ref[...] = c * (ew < CUTOFF_UPPER).astype(jnp.float32)


def _cutoff(edge_weight):
    ew2 = edge_weight.reshape(E // H, H)
    return pl.pallas_call(
        _cutoff_body,
        out_shape=jax.ShapeDtypeStruct((E // H, H), jnp.float32),
    )(ew2)


def _edge_matmul_body(eat_ref, w1_ref, b1_ref, cw_ref, out_ref):
    w = lax.dot_general(eat_ref[...], w1_ref[...],
                        (((0,), (0,)), ((), ())),
                        preferred_element_type=jnp.float32)
    w = w + b1_ref[...]
    c = cw_ref[0]
    w3 = w.reshape(BE // H, H, H) * c[:, :, None]
    out_ref[...] = w3.reshape(BE, H)


def _edge_features(ea_t, W1, b1, cw3, h):
    off = h * (EH // BE)
    return pl.pallas_call(
        _edge_matmul_body,
        grid=(EH // BE,),
        in_specs=[
            pl.BlockSpec((D_EDGE, BE), lambda i: (0, i + off)),
            pl.BlockSpec((D_EDGE, H), lambda i: (0, 0)),
            pl.BlockSpec((1, H), lambda i: (0, 0)),
            pl.BlockSpec((1, BE // H, H), lambda i: (i + off, 0, 0)),
        ],
        out_specs=pl.BlockSpec((BE, H), lambda i: (i, 0)),
        out_shape=jax.ShapeDtypeStruct((EH, H), jnp.float32),
    )(ea_t, W1, b1.reshape(1, H), cw3)


def _embed_body(z_ref, emb_ref, out_ref):
    zb = z_ref[...]
    iota = lax.broadcasted_iota(jnp.int32, (BN, MAX_SPECIES), 1)
    oh = (zb == iota).astype(jnp.float32)
    out_ref[...] = jnp.dot(oh, emb_ref[...], preferred_element_type=jnp.float32)


def _embed_lookup(z, embed):
    z2 = z.astype(jnp.int32).reshape(N, 1)
    return pl.pallas_call(
        _embed_body,
        grid=(N // BN,),
        in_specs=[
            pl.BlockSpec((BN, 1), lambda i: (i, 0)),
            pl.BlockSpec((MAX_SPECIES, H), lambda i: (0, 0)),
        ],
        out_specs=pl.BlockSpec((BN, H), lambda i: (i, 0)),
        out_shape=jax.ShapeDtypeStruct((N, H), jnp.float32),
    )(z2, embed)


NB = 4


def _sc_scatter_body(h, wc_hbm, xn_hbm, snd_hbm, rcv_hbm, out_hbm,
                     *sc):
    sidx = sc[0:NB]
    ridx = sc[NB:2 * NB]
    wcv = sc[2 * NB:3 * NB]
    xsv = sc[3 * NB:4 * NB]
    aggsh = sc[4 * NB]
    sem_cp = sc[4 * NB + 1:5 * NB + 1]
    sem_g = sc[5 * NB + 1:6 * NB + 1]
    sem_sc = sc[6 * NB + 1:7 * NB + 1]

    c = lax.axis_index("c")
    s = lax.axis_index("s")
    wid = s * 2 + c
    tile_base = h * EH + wid * (JPT * CH)

    zv = jnp.zeros((L,), jnp.float32)

    def _zero_row(i, carry):
        for cc in range(H // L):
            xsv[0][i, pl.ds(cc * L, L)] = zv
        return carry

    lax.fori_loop(0, CH, _zero_row, 0)
    row0 = s * ROWS_PER_TILE
    for k in range(ROWS_PER_TILE // CH):
        pltpu.sync_copy(xsv[0], aggsh.at[pl.ds(row0 + k * CH, CH)])
    rem = ROWS_PER_TILE % CH
    if rem:
        pltpu.sync_copy(xsv[0].at[pl.ds(0, rem)],
                        aggsh.at[pl.ds(row0 + ROWS_PER_TILE - rem, rem)])

    @pl.when(s == 15)
    def _zero_tail():
        pltpu.sync_copy(xsv[0].at[pl.ds(0, N - 16 * ROWS_PER_TILE)],
                        aggsh.at[pl.ds(16 * ROWS_PER_TILE,
                                       N - 16 * ROWS_PER_TILE)])

    plsc.subcore_barrier()

    def s1(j, X):
        base = tile_base + j * CH
        wbase = base - h * EH
        pltpu.async_copy(snd_hbm.at[pl.ds(base, CH)], sidx[X], sem_cp[X])
        pltpu.async_copy(wc_hbm.at[pl.ds(wbase, CH)], wcv[X], sem_cp[X])

    def s2(j, X, first):
        if not first:
            pltpu.make_async_copy(xsv[X], aggsh.at[ridx[X]], sem_sc[X]).wait()
        pltpu.make_async_copy(snd_hbm.at[pl.ds(0, CH)], sidx[X],
                              sem_cp[X]).wait()
        pltpu.make_async_copy(wc_hbm.at[pl.ds(0, CH)], wcv[X],
                              sem_cp[X]).wait()
        base = tile_base + j * CH
        pltpu.async_copy(rcv_hbm.at[pl.ds(base, CH)], ridx[X], sem_g[X])
        pltpu.async_copy(xn_hbm.at[sidx[X]], xsv[X], sem_g[X])

    def _mul(X):
        @plsc.parallel_loop(0, CH, 1, unroll=2)
        def _mul_row(i):
            for g in range(H // L):
                sl = pl.ds(g * L, L)
                xsv[X][i, sl] = xsv[X][i, sl] * wcv[X][i, sl]

    def s3(j, X, last):
        pltpu.make_async_copy(rcv_hbm.at[pl.ds(0, CH)], ridx[X],
                              sem_g[X]).wait()
        pltpu.make_async_copy(xn_hbm.at[sidx[X]], xsv[X], sem_g[X]).wait()
        _mul(X)
        if last:
            pltpu.sync_copy(xsv[X], aggsh.at[ridx[X]], add=True)
        else:
            pltpu.async_copy(xsv[X], aggsh.at[ridx[X]], sem_sc[X], add=True)

    for t in range(NB):
        s1(t, t)
    for t in range(NB):
        s2(t, t, True)
        s3(t, t, False)
        s1(t + NB, t)

    def _steady(k, carry):
        for t in range(NB):
            j = NB * k + t
            s2(j, t, False)
            s3(j, t, False)
            s1(j + NB, t)
        return carry

    assert JPT >= 3 * NB
    k_end = (JPT - 2 * NB) // NB
    lax.fori_loop(1, k_end + 1, _steady, 0)

    for j in range(NB * (k_end + 1), JPT):
        X = j % NB
        s2(j, X, False)
        s3(j, X, j + NB >= JPT)
        if j + NB < JPT:
            s1(j + NB, X)

    plsc.subcore_barrier()

    pltpu.sync_copy(aggsh.at[pl.ds(row0, ROWS_PER_TILE)],
                    out_hbm.at[c, pl.ds(row0, ROWS_PER_TILE)])

    @pl.when(s == 15)
    def _write_tail():
        pltpu.sync_copy(aggsh.at[pl.ds(16 * ROWS_PER_TILE,
                                       N - 16 * ROWS_PER_TILE)],
                        out_hbm.at[c, pl.ds(16 * ROWS_PER_TILE,
                                            N - 16 * ROWS_PER_TILE)])


def _sc_scatter(wc, xn, senders, receivers, h):
    mesh = plsc.VectorSubcoreMesh(core_axis_name="c", subcore_axis_name="s")
    f = pl.kernel(
        functools.partial(_sc_scatter_body, h),
        out_type=jax.ShapeDtypeStruct((2, N, H), jnp.float32),
        mesh=mesh,
        compiler_params=pltpu.CompilerParams(needs_layout_passes=False),
        scratch_types=(
            [pltpu.VMEM((CH,), jnp.int32) for _ in range(NB)]
            + [pltpu.VMEM((CH,), jnp.int32) for _ in range(NB)]
            + [pltpu.VMEM((CH, H), jnp.float32) for _ in range(NB)]
            + [pltpu.VMEM((CH, H), jnp.float32) for _ in range(NB)]
            + [pltpu.VMEM_SHARED((N, H), jnp.float32)]
            + [pltpu.SemaphoreType.DMA for _ in range(3 * NB)]
        ),
    )
    return f(wc, xn, senders, receivers)


def _out_body(x_ref, pa_ref, pb_ref, w2a_ref, w2b_ref, b2_ref, o_ref):
    agg = (pa_ref[0] + pa_ref[1]) + (pb_ref[0] + pb_ref[1])
    o_ref[...] = (
        jnp.dot(x_ref[...], w2a_ref[...], preferred_element_type=jnp.float32)
        + jnp.dot(agg, w2b_ref[...], preferred_element_type=jnp.float32)
        + b2_ref[...]
    )


def _out_proj(x, pa, pb, W2, b2):
    return pl.pallas_call(
        _out_body,
        grid=(N // BN,),
        in_specs=[
            pl.BlockSpec((BN, H), lambda i: (i, 0)),
            pl.BlockSpec((2, BN, H), lambda i: (0, i, 0)),
            pl.BlockSpec((2, BN, H), lambda i: (0, i, 0)),
            pl.BlockSpec((H, H), lambda i: (0, 0)),
            pl.BlockSpec((H, H), lambda i: (0, 0)),
            pl.BlockSpec((1, H), lambda i: (0, 0)),
        ],
        out_specs=pl.BlockSpec((BN, H), lambda i: (i, 0)),
        out_shape=jax.ShapeDtypeStruct((N, H), jnp.float32),
    )(x, pa, pb, W2[:H], W2[H:], b2.reshape(1, H))


def kernel(z, x, senders, receivers, edge_weight, edge_attr, W1, b1, embed, W2, b2):
    senders = senders.astype(jnp.int32)
    receivers = receivers.astype(jnp.int32)
    cw2 = _cutoff(edge_weight)
    cw3 = cw2.reshape(E // BE, BE // H, H)
    ea_t = edge_attr.T
    xn = _embed_lookup(z, embed)
    wc0 = _edge_features(ea_t, W1, b1, cw3, 0)
    pa = _sc_scatter(wc0, xn, senders, receivers, 0)
    wc1 = _edge_features(ea_t, W1, b1, cw3, 1)
    pb = _sc_scatter(wc1, xn, senders, receivers, 1)
    return _out_proj(x, pa, pb, W2, b2)


# speedup vs baseline: 6.1235x; 1.4273x over previous
import functools
import math

import numpy as np
import jax
import jax.numpy as jnp
from jax import lax
from jax.experimental import pallas as pl
from jax.experimental.pallas import tpu as pltpu
from jax.experimental.pallas import tpu_sc as plsc

N = 10000
E = 320000
H = 128
D_EDGE = 16
MAX_SPECIES = 100
CUTOFF_UPPER = 5.0

L = 16
NW = 32
CH = 40
NH = 2
EH = E // NH
JPT = EH // (NW * CH)
ROWS_PER_TILE = 624

BE = 6400
BN = 2000


def _cutoff_body(ew_ref, out_ref):
    ew = ew_ref[...]
    c = 0.5 * (jnp.cos(ew * (math.pi / CUTOFF_UPPER)) + 1.0)
    out_ref[...] = c * (ew < CUTOFF_UPPER).astype(jnp.float32)


def _cutoff(edge_weight):
    ew2 = edge_weight.reshape(E // H, H)
    return pl.pallas_call(
        _cutoff_body,
        out_shape=jax.ShapeDtypeStruct((E // H, H), jnp.float32),
    )(ew2)


def _edge_matmul_body(eat_ref, w1_ref, b1_ref, cw_ref, out_ref):
    w = lax.dot_general(eat_ref[...], w1_ref[...],
                        (((0,), (0,)), ((), ())),
                        preferred_element_type=jnp.float32)
    w = w + b1_ref[...]
    c = cw_ref[0]
    w3 = w.reshape(BE // H, H, H) * c[:, :, None]
    out_ref[...] = w3.reshape(BE, H)


def _edge_features(ea_t, W1, b1, cw3, h):
    off = h * (EH // BE)
    return pl.pallas_call(
        _edge_matmul_body,
        grid=(EH // BE,),
        in_specs=[
            pl.BlockSpec((D_EDGE, BE), lambda i: (0, i + off)),
            pl.BlockSpec((D_EDGE, H), lambda i: (0, 0)),
            pl.BlockSpec((1, H), lambda i: (0, 0)),
            pl.BlockSpec((1, BE // H, H), lambda i: (i + off, 0, 0)),
        ],
        out_specs=pl.BlockSpec((BE, H), lambda i: (i, 0)),
        out_shape=jax.ShapeDtypeStruct((EH, H), jnp.float32),
    )(ea_t, W1, b1.reshape(1, H), cw3)


def _embed_body(z_ref, emb_ref, out_ref):
    zb = z_ref[...]
    iota = lax.broadcasted_iota(jnp.int32, (BN, MAX_SPECIES), 1)
    oh = (zb == iota).astype(jnp.float32)
    out_ref[...] = jnp.dot(oh, emb_ref[...], preferred_element_type=jnp.float32)


def _embed_lookup(z, embed):
    z2 = z.astype(jnp.int32).reshape(N, 1)
    return pl.pallas_call(
        _embed_body,
        grid=(N // BN,),
        in_specs=[
            pl.BlockSpec((BN, 1), lambda i: (i, 0)),
            pl.BlockSpec((MAX_SPECIES, H), lambda i: (0, 0)),
        ],
        out_specs=pl.BlockSpec((BN, H), lambda i: (i, 0)),
        out_shape=jax.ShapeDtypeStruct((N, H), jnp.float32),
    )(z2, embed)


NB = 4


def _sc_scatter_body(h, wc_hbm, xn_hbm, snd_hbm, rcv_hbm, out_hbm,
                     *sc):
    sidx = sc[0:NB]
    ridx = sc[NB:2 * NB]
    wcv = sc[2 * NB:3 * NB]
    xsv = sc[3 * NB:4 * NB]
    aggsh = sc[4 * NB]
    sem_cp = sc[4 * NB + 1:5 * NB + 1]
    sem_g = sc[5 * NB + 1:6 * NB + 1]
    sem_sc = sc[6 * NB + 1:7 * NB + 1]

    c = lax.axis_index("c")
    s = lax.axis_index("s")
    wid = s * 2 + c
    tile_base = h * EH + wid * (JPT * CH)

    zv = jnp.zeros((L,), jnp.float32)

    def _zero_row(i, carry):
        for cc in range(H // L):
            xsv[0][i, pl.ds(cc * L, L)] = zv
        return carry

    lax.fori_loop(0, CH, _zero_row, 0)
    row0 = s * ROWS_PER_TILE
    for k in range(ROWS_PER_TILE // CH):
        pltpu.sync_copy(xsv[0], aggsh.at[pl.ds(row0 + k * CH, CH)])
    rem = ROWS_PER_TILE % CH
    if rem:
        pltpu.sync_copy(xsv[0].at[pl.ds(0, rem)],
                        aggsh.at[pl.ds(row0 + ROWS_PER_TILE - rem, rem)])

    @pl.when(s == 15)
    def _zero_tail():
        pltpu.sync_copy(xsv[0].at[pl.ds(0, N - 16 * ROWS_PER_TILE)],
                        aggsh.at[pl.ds(16 * ROWS_PER_TILE,
                                       N - 16 * ROWS_PER_TILE)])

    plsc.subcore_barrier()

    def s1(j, X):
        base = tile_base + j * CH
        wbase = base - h * EH
        pltpu.async_copy(snd_hbm.at[pl.ds(base, CH)], sidx[X], sem_cp[X])
        pltpu.async_copy(wc_hbm.at[pl.ds(wbase, CH)], wcv[X], sem_cp[X])

    def s2(j, X, first):
        if not first:
            pltpu.make_async_copy(xsv[X], aggsh.at[ridx[X]], sem_sc[X]).wait()
        pltpu.make_async_copy(snd_hbm.at[pl.ds(0, CH)], sidx[X],
                              sem_cp[X]).wait()
        pltpu.make_async_copy(wc_hbm.at[pl.ds(0, CH)], wcv[X],
                              sem_cp[X]).wait()
        base = tile_base + j * CH
        pltpu.async_copy(rcv_hbm.at[pl.ds(base, CH)], ridx[X], sem_g[X])
        pltpu.async_copy(xn_hbm.at[sidx[X]], xsv[X], sem_g[X])

    def _mul(X):
        @plsc.parallel_loop(0, CH, 1, unroll=2)
        def _mul_row(i):
            for g in range(H // L):
                sl = pl.ds(g * L, L)
                xsv[X][i, sl] = xsv[X][i, sl] * wcv[X][i, sl]

    def s3(j, X, last):
        pltpu.make_async_copy(rcv_hbm.at[pl.ds(0, CH)], ridx[X],
                              sem_g[X]).wait()
        pltpu.make_async_copy(xn_hbm.at[sidx[X]], xsv[X], sem_g[X]).wait()
        _mul(X)
        if last:
            pltpu.sync_copy(xsv[X], aggsh.at[ridx[X]], add=True)
        else:
            pltpu.async_copy(xsv[X], aggsh.at[ridx[X]], sem_sc[X], add=True)

    for t in range(NB):
        s1(t, t)
    s2(0, 0, True)
    for j in range(NB):
        s2(j + 1, (j + 1) % NB, j + 1 < NB)
        s3(j, j % NB, False)
        s1(j + NB, j % NB)

    def _steady(k, carry):
        for t in range(NB):
            j = NB * k + t
            s2(j + 1, (t + 1) % NB, False)
            s3(j, t, False)
            s1(j + NB, t)
        return carry

    assert JPT >= 3 * NB
    k_end = (JPT - 2 * NB) // NB
    lax.fori_loop(1, k_end + 1, _steady, 0)

    for j in range(NB * (k_end + 1), JPT):
        X = j % NB
        if j + 1 < JPT:
            s2(j + 1, (j + 1) % NB, False)
        s3(j, X, j + NB >= JPT)
        if j + NB < JPT:
            s1(j + NB, X)

    plsc.subcore_barrier()

    pltpu.sync_copy(aggsh.at[pl.ds(row0, ROWS_PER_TILE)],
                    out_hbm.at[c, pl.ds(row0, ROWS_PER_TILE)])

    @pl.when(s == 15)
    def _write_tail():
        pltpu.sync_copy(aggsh.at[pl.ds(16 * ROWS_PER_TILE,
                                       N - 16 * ROWS_PER_TILE)],
                        out_hbm.at[c, pl.ds(16 * ROWS_PER_TILE,
                                            N - 16 * ROWS_PER_TILE)])


def _sc_scatter(wc, xn, senders, receivers, h):
    mesh = plsc.VectorSubcoreMesh(core_axis_name="c", subcore_axis_name="s")
    f = pl.kernel(
        functools.partial(_sc_scatter_body, h),
        out_type=jax.ShapeDtypeStruct((2, N, H), jnp.float32),
        mesh=mesh,
        compiler_params=pltpu.CompilerParams(needs_layout_passes=False),
        scratch_types=(
            [pltpu.VMEM((CH,), jnp.int32) for _ in range(NB)]
            + [pltpu.VMEM((CH,), jnp.int32) for _ in range(NB)]
            + [pltpu.VMEM((CH, H), jnp.float32) for _ in range(NB)]
            + [pltpu.VMEM((CH, H), jnp.float32) for _ in range(NB)]
            + [pltpu.VMEM_SHARED((N, H), jnp.float32)]
            + [pltpu.SemaphoreType.DMA for _ in range(3 * NB)]
        ),
    )
    return f(wc, xn, senders, receivers)


def _out_body(x_ref, pa_ref, pb_ref, w2a_ref, w2b_ref, b2_ref, o_ref):
    agg = (pa_ref[0] + pa_ref[1]) + (pb_ref[0] + pb_ref[1])
    o_ref[...] = (
        jnp.dot(x_ref[...], w2a_ref[...], preferred_element_type=jnp.float32)
        + jnp.dot(agg, w2b_ref[...], preferred_element_type=jnp.float32)
        + b2_ref[...]
    )


def _out_proj(x, pa, pb, W2, b2):
    return pl.pallas_call(
        _out_body,
        grid=(N // BN,),
        in_specs=[
            pl.BlockSpec((BN, H), lambda i: (i, 0)),
            pl.BlockSpec((2, BN, H), lambda i: (0, i, 0)),
            pl.BlockSpec((2, BN, H), lambda i: (0, i, 0)),
            pl.BlockSpec((H, H), lambda i: (0, 0)),
            pl.BlockSpec((H, H), lambda i: (0, 0)),
            pl.BlockSpec((1, H), lambda i: (0, 0)),
        ],
        out_specs=pl.BlockSpec((BN, H), lambda i: (i, 0)),
        out_shape=jax.ShapeDtypeStruct((N, H), jnp.float32),
    )(x, pa, pb, W2[:H], W2[H:], b2.reshape(1, H))


def kernel(z, x, senders, receivers, edge_weight, edge_attr, W1, b1, embed, W2, b2):
    senders = senders.astype(jnp.int32)
    receivers = receivers.astype(jnp.int32)
    cw2 = _cutoff(edge_weight)
    cw3 = cw2.reshape(E // BE, BE // H, H)
    ea_t = edge_attr.T
    xn = _embed_lookup(z, embed)
    wc0 = _edge_features(ea_t, W1, b1, cw3, 0)
    pa = _sc_scatter(wc0, xn, senders, receivers, 0)
    wc1 = _edge_features(ea_t, W1, b1, cw3, 1)
    pb = _sc_scatter(wc1, xn, senders, receivers, 1)
    return _out_proj(x, pa, pb, W2, b2)


# speedup vs baseline: 6.1376x; 1.0023x over previous
import functools
import math

import numpy as np
import jax
import jax.numpy as jnp
from jax import lax
from jax.experimental import pallas as pl
from jax.experimental.pallas import tpu as pltpu
from jax.experimental.pallas import tpu_sc as plsc

N = 10000
E = 320000
H = 128
D_EDGE = 16
MAX_SPECIES = 100
CUTOFF_UPPER = 5.0

L = 16
NW = 32
CH = 40
NH = 2
EH = E // NH
JPT = EH // (NW * CH)
ROWS_PER_TILE = 624

BE = 6400
BN = 2000


def _cutoff_body(ew_ref, out_ref):
    ew = ew_ref[...]
    c = 0.5 * (jnp.cos(ew * (math.pi / CUTOFF_UPPER)) + 1.0)
    out_ref[...] = c * (ew < CUTOFF_UPPER).astype(jnp.float32)


def _cutoff(edge_weight):
    ew2 = edge_weight.reshape(E // H, H)
    return pl.pallas_call(
        _cutoff_body,
        out_shape=jax.ShapeDtypeStruct((E // H, H), jnp.float32),
    )(ew2)


def _edge_matmul_body(eat_ref, w1_ref, b1_ref, cw_ref, out_ref):
    w = lax.dot_general(eat_ref[...], w1_ref[...],
                        (((0,), (0,)), ((), ())),
                        preferred_element_type=jnp.float32)
    w = w + b1_ref[...]
    c = cw_ref[0]
    w3 = w.reshape(BE // H, H, H) * c[:, :, None]
    out_ref[...] = w3.reshape(BE, H)


def _edge_features(ea_t, W1, b1, cw3, h):
    off = h * (EH // BE)
    return pl.pallas_call(
        _edge_matmul_body,
        grid=(EH // BE,),
        in_specs=[
            pl.BlockSpec((D_EDGE, BE), lambda i: (0, i + off)),
            pl.BlockSpec((D_EDGE, H), lambda i: (0, 0)),
            pl.BlockSpec((1, H), lambda i: (0, 0)),
            pl.BlockSpec((1, BE // H, H), lambda i: (i + off, 0, 0)),
        ],
        out_specs=pl.BlockSpec((BE, H), lambda i: (i, 0)),
        out_shape=jax.ShapeDtypeStruct((EH, H), jnp.float32),
    )(ea_t, W1, b1.reshape(1, H), cw3)


def _embed_body(z_ref, emb_ref, out_ref):
    zb = z_ref[...]
    iota = lax.broadcasted_iota(jnp.int32, (BN, MAX_SPECIES), 1)
    oh = (zb == iota).astype(jnp.float32)
    out_ref[...] = jnp.dot(oh, emb_ref[...], preferred_element_type=jnp.float32)


def _embed_lookup(z, embed):
    z2 = z.astype(jnp.int32).reshape(N, 1)
    return pl.pallas_call(
        _embed_body,
        grid=(N // BN,),
        in_specs=[
            pl.BlockSpec((BN, 1), lambda i: (i, 0)),
            pl.BlockSpec((MAX_SPECIES, H), lambda i: (0, 0)),
        ],
        out_specs=pl.BlockSpec((BN, H), lambda i: (i, 0)),
        out_shape=jax.ShapeDtypeStruct((N, H), jnp.float32),
    )(z2, embed)


NB = 4


def _sc_scatter_body(h, wc_hbm, xn_hbm, snd_hbm, rcv_hbm, out_hbm,
                     *sc):
    sidx = sc[0:NB]
    ridx = sc[NB:2 * NB]
    wcv = sc[2 * NB:3 * NB]
    xsv = sc[3 * NB:4 * NB]
    aggsh = sc[4 * NB]
    sem_cp = sc[4 * NB + 1:5 * NB + 1]
    sem_g = sc[5 * NB + 1:6 * NB + 1]
    sem_sc = sc[6 * NB + 1:7 * NB + 1]

    c = lax.axis_index("c")
    s = lax.axis_index("s")
    wid = s * 2 + c
    tile_base = h * EH + wid * (JPT * CH)

    zv = jnp.zeros((L,), jnp.float32)

    def _zero_row(i, carry):
        for cc in range(H // L):
            xsv[0][i, pl.ds(cc * L, L)] = zv
        return carry

    lax.fori_loop(0, CH, _zero_row, 0)
    row0 = s * ROWS_PER_TILE
    for k in range(ROWS_PER_TILE // CH):
        pltpu.sync_copy(xsv[0], aggsh.at[pl.ds(row0 + k * CH, CH)])
    rem = ROWS_PER_TILE % CH
    if rem:
        pltpu.sync_copy(xsv[0].at[pl.ds(0, rem)],
                        aggsh.at[pl.ds(row0 + ROWS_PER_TILE - rem, rem)])

    @pl.when(s == 15)
    def _zero_tail():
        pltpu.sync_copy(xsv[0].at[pl.ds(0, N - 16 * ROWS_PER_TILE)],
                        aggsh.at[pl.ds(16 * ROWS_PER_TILE,
                                       N - 16 * ROWS_PER_TILE)])

    plsc.subcore_barrier()

    def s1(j, X):
        base = tile_base + j * CH
        wbase = base - h * EH
        pltpu.async_copy(snd_hbm.at[pl.ds(base, CH)], sidx[X], sem_cp[X])
        pltpu.async_copy(wc_hbm.at[pl.ds(wbase, CH)], wcv[X], sem_cp[X])

    def s2(j, X, first):
        if not first:
            pltpu.make_async_copy(xsv[X], aggsh.at[ridx[X]], sem_sc[X]).wait()
        pltpu.make_async_copy(snd_hbm.at[pl.ds(0, CH)], sidx[X],
                              sem_cp[X]).wait()
        pltpu.make_async_copy(wc_hbm.at[pl.ds(0, CH)], wcv[X],
                              sem_cp[X]).wait()
        base = tile_base + j * CH
        pltpu.async_copy(rcv_hbm.at[pl.ds(base, CH)], ridx[X], sem_g[X])
        pltpu.async_copy(xn_hbm.at[sidx[X]], xsv[X], sem_g[X])

    def _mul(X):
        @plsc.parallel_loop(0, CH, 1, unroll=4)
        def _mul_row(i):
            for g in range(H // L):
                sl = pl.ds(g * L, L)
                xsv[X][i, sl] = xsv[X][i, sl] * wcv[X][i, sl]

    def s3(j, X, last):
        pltpu.make_async_copy(rcv_hbm.at[pl.ds(0, CH)], ridx[X],
                              sem_g[X]).wait()
        pltpu.make_async_copy(xn_hbm.at[sidx[X]], xsv[X], sem_g[X]).wait()
        _mul(X)
        if last:
            pltpu.sync_copy(xsv[X], aggsh.at[ridx[X]], add=True)
        else:
            pltpu.async_copy(xsv[X], aggsh.at[ridx[X]], sem_sc[X], add=True)

    for t in range(NB):
        s1(t, t)
    s2(0, 0, True)
    for j in range(NB):
        s2(j + 1, (j + 1) % NB, j + 1 < NB)
        s3(j, j % NB, False)
        s1(j + NB, j % NB)

    def _steady(k, carry):
        for t in range(NB):
            j = NB * k + t
            s2(j + 1, (t + 1) % NB, False)
            s3(j, t, False)
            s1(j + NB, t)
        return carry

    assert JPT >= 3 * NB
    k_end = (JPT - 2 * NB) // NB
    lax.fori_loop(1, k_end + 1, _steady, 0)

    for j in range(NB * (k_end + 1), JPT):
        X = j % NB
        if j + 1 < JPT:
            s2(j + 1, (j + 1) % NB, False)
        s3(j, X, j + NB >= JPT)
        if j + NB < JPT:
            s1(j + NB, X)

    plsc.subcore_barrier()

    pltpu.sync_copy(aggsh.at[pl.ds(row0, ROWS_PER_TILE)],
                    out_hbm.at[c, pl.ds(row0, ROWS_PER_TILE)])

    @pl.when(s == 15)
    def _write_tail():
        pltpu.sync_copy(aggsh.at[pl.ds(16 * ROWS_PER_TILE,
                                       N - 16 * ROWS_PER_TILE)],
                        out_hbm.at[c, pl.ds(16 * ROWS_PER_TILE,
                                            N - 16 * ROWS_PER_TILE)])


def _sc_scatter(wc, xn, senders, receivers, h):
    mesh = plsc.VectorSubcoreMesh(core_axis_name="c", subcore_axis_name="s")
    f = pl.kernel(
        functools.partial(_sc_scatter_body, h),
        out_type=jax.ShapeDtypeStruct((2, N, H), jnp.float32),
        mesh=mesh,
        compiler_params=pltpu.CompilerParams(needs_layout_passes=False),
        scratch_types=(
            [pltpu.VMEM((CH,), jnp.int32) for _ in range(NB)]
            + [pltpu.VMEM((CH,), jnp.int32) for _ in range(NB)]
            + [pltpu.VMEM((CH, H), jnp.float32) for _ in range(NB)]
            + [pltpu.VMEM((CH, H), jnp.float32) for _ in range(NB)]
            + [pltpu.VMEM_SHARED((N, H), jnp.float32)]
            + [pltpu.SemaphoreType.DMA for _ in range(3 * NB)]
        ),
    )
    return f(wc, xn, senders, receivers)


def _out_body(x_ref, pa_ref, pb_ref, w2a_ref, w2b_ref, b2_ref, o_ref):
    agg = (pa_ref[0] + pa_ref[1]) + (pb_ref[0] + pb_ref[1])
    o_ref[...] = (
        jnp.dot(x_ref[...], w2a_ref[...], preferred_element_type=jnp.float32)
        + jnp.dot(agg, w2b_ref[...], preferred_element_type=jnp.float32)
        + b2_ref[...]
    )


def _out_proj(x, pa, pb, W2, b2):
    return pl.pallas_call(
        _out_body,
        grid=(N // BN,),
        in_specs=[
            pl.BlockSpec((BN, H), lambda i: (i, 0)),
            pl.BlockSpec((2, BN, H), lambda i: (0, i, 0)),
            pl.BlockSpec((2, BN, H), lambda i: (0, i, 0)),
            pl.BlockSpec((H, H), lambda i: (0, 0)),
            pl.BlockSpec((H, H), lambda i: (0, 0)),
            pl.BlockSpec((1, H), lambda i: (0, 0)),
        ],
        out_specs=pl.BlockSpec((BN, H), lambda i: (i, 0)),
        out_shape=jax.ShapeDtypeStruct((N, H), jnp.float32),
    )(x, pa, pb, W2[:H], W2[H:], b2.reshape(1, H))


def kernel(z, x, senders, receivers, edge_weight, edge_attr, W1, b1, embed, W2, b2):
    senders = senders.astype(jnp.int32)
    receivers = receivers.astype(jnp.int32)
    cw2 = _cutoff(edge_weight)
    cw3 = cw2.reshape(E // BE, BE // H, H)
    ea_t = edge_attr.T
    xn = _embed_lookup(z, embed)
    wc0 = _edge_features(ea_t, W1, b1, cw3, 0)
    pa = _sc_scatter(wc0, xn, senders, receivers, 0)
    wc1 = _edge_features(ea_t, W1, b1, cw3, 1)
    pb = _sc_scatter(wc1, xn, senders, receivers, 1)
    return _out_proj(x, pa, pb, W2, b2)


# speedup vs baseline: 6.2383x; 1.0164x over previous
import functools
import math

import numpy as np
import jax
import jax.numpy as jnp
from jax import lax
from jax.experimental import pallas as pl
from jax.experimental.pallas import tpu as pltpu
from jax.experimental.pallas import tpu_sc as plsc

N = 10000
E = 320000
H = 128
D_EDGE = 16
MAX_SPECIES = 100
CUTOFF_UPPER = 5.0

L = 16
NW = 32
CH = 40
NH = 2
EH = E // NH
JPT = EH // (NW * CH)
ROWS_PER_TILE = 624

BE = 6400
BN = 2000


def _edge_matmul_body(eat_ref, w1_ref, b1_ref, ew_ref, out_ref):
    w = lax.dot_general(eat_ref[...], w1_ref[...],
                        (((0,), (0,)), ((), ())),
                        preferred_element_type=jnp.float32)
    w = w + b1_ref[...]
    ew = ew_ref[0]
    c = 0.5 * (jnp.cos(ew * (math.pi / CUTOFF_UPPER)) + 1.0)
    c = c * (ew < CUTOFF_UPPER).astype(jnp.float32)
    w3 = w.reshape(BE // H, H, H) * c[:, :, None]
    out_ref[...] = w3.reshape(BE, H)


def _edge_features(ea_t, W1, b1, ew3, h):
    off = h * (EH // BE)
    return pl.pallas_call(
        _edge_matmul_body,
        grid=(EH // BE,),
        in_specs=[
            pl.BlockSpec((D_EDGE, BE), lambda i: (0, i + off)),
            pl.BlockSpec((D_EDGE, H), lambda i: (0, 0)),
            pl.BlockSpec((1, H), lambda i: (0, 0)),
            pl.BlockSpec((1, BE // H, H), lambda i: (i + off, 0, 0)),
        ],
        out_specs=pl.BlockSpec((BE, H), lambda i: (i, 0)),
        out_shape=jax.ShapeDtypeStruct((EH, H), jnp.float32),
    )(ea_t, W1, b1.reshape(1, H), ew3)


def _embed_body(z_ref, emb_ref, out_ref):
    zb = z_ref[...]
    iota = lax.broadcasted_iota(jnp.int32, (BN, MAX_SPECIES), 1)
    oh = (zb == iota).astype(jnp.float32)
    out_ref[...] = jnp.dot(oh, emb_ref[...], preferred_element_type=jnp.float32)


def _embed_lookup(z, embed):
    z2 = z.astype(jnp.int32).reshape(N, 1)
    return pl.pallas_call(
        _embed_body,
        grid=(N // BN,),
        in_specs=[
            pl.BlockSpec((BN, 1), lambda i: (i, 0)),
            pl.BlockSpec((MAX_SPECIES, H), lambda i: (0, 0)),
        ],
        out_specs=pl.BlockSpec((BN, H), lambda i: (i, 0)),
        out_shape=jax.ShapeDtypeStruct((N, H), jnp.float32),
    )(z2, embed)


NB = 4


def _sc_scatter_body(h, wc_hbm, xn_hbm, snd_hbm, rcv_hbm, out_hbm,
                     *sc):
    sidx = sc[0:NB]
    ridx = sc[NB:2 * NB]
    wcv = sc[2 * NB:3 * NB]
    xsv = sc[3 * NB:4 * NB]
    aggsh = sc[4 * NB]
    sem_cp = sc[4 * NB + 1:5 * NB + 1]
    sem_g = sc[5 * NB + 1:6 * NB + 1]
    sem_sc = sc[6 * NB + 1:7 * NB + 1]

    c = lax.axis_index("c")
    s = lax.axis_index("s")
    wid = s * 2 + c
    tile_base = h * EH + wid * (JPT * CH)

    zv = jnp.zeros((L,), jnp.float32)

    def _zero_row(i, carry):
        for cc in range(H // L):
            xsv[0][i, pl.ds(cc * L, L)] = zv
        return carry

    lax.fori_loop(0, CH, _zero_row, 0)
    row0 = s * ROWS_PER_TILE
    for k in range(ROWS_PER_TILE // CH):
        pltpu.sync_copy(xsv[0], aggsh.at[pl.ds(row0 + k * CH, CH)])
    rem = ROWS_PER_TILE % CH
    if rem:
        pltpu.sync_copy(xsv[0].at[pl.ds(0, rem)],
                        aggsh.at[pl.ds(row0 + ROWS_PER_TILE - rem, rem)])

    @pl.when(s == 15)
    def _zero_tail():
        pltpu.sync_copy(xsv[0].at[pl.ds(0, N - 16 * ROWS_PER_TILE)],
                        aggsh.at[pl.ds(16 * ROWS_PER_TILE,
                                       N - 16 * ROWS_PER_TILE)])

    plsc.subcore_barrier()

    def s1(j, X):
        base = tile_base + j * CH
        wbase = base - h * EH
        pltpu.async_copy(snd_hbm.at[pl.ds(base, CH)], sidx[X], sem_cp[X])
        pltpu.async_copy(wc_hbm.at[pl.ds(wbase, CH)], wcv[X], sem_cp[X])

    def s2(j, X, first):
        if not first:
            pltpu.make_async_copy(xsv[X], aggsh.at[ridx[X]], sem_sc[X]).wait()
        pltpu.make_async_copy(snd_hbm.at[pl.ds(0, CH)], sidx[X],
                              sem_cp[X]).wait()
        pltpu.make_async_copy(wc_hbm.at[pl.ds(0, CH)], wcv[X],
                              sem_cp[X]).wait()
        base = tile_base + j * CH
        pltpu.async_copy(rcv_hbm.at[pl.ds(base, CH)], ridx[X], sem_g[X])
        pltpu.async_copy(xn_hbm.at[sidx[X]], xsv[X], sem_g[X])

    def _mul(X):
        @plsc.parallel_loop(0, CH, 1, unroll=4)
        def _mul_row(i):
            for g in range(H // L):
                sl = pl.ds(g * L, L)
                xsv[X][i, sl] = xsv[X][i, sl] * wcv[X][i, sl]

    def s3(j, X, last):
        pltpu.make_async_copy(rcv_hbm.at[pl.ds(0, CH)], ridx[X],
                              sem_g[X]).wait()
        pltpu.make_async_copy(xn_hbm.at[sidx[X]], xsv[X], sem_g[X]).wait()
        _mul(X)
        if last:
            pltpu.sync_copy(xsv[X], aggsh.at[ridx[X]], add=True)
        else:
            pltpu.async_copy(xsv[X], aggsh.at[ridx[X]], sem_sc[X], add=True)

    for t in range(NB):
        s1(t, t)
    s2(0, 0, True)
    for j in range(NB):
        s2(j + 1, (j + 1) % NB, j + 1 < NB)
        s3(j, j % NB, False)
        s1(j + NB, j % NB)

    def _steady(k, carry):
        for t in range(NB):
            j = NB * k + t
            s2(j + 1, (t + 1) % NB, False)
            s3(j, t, False)
            s1(j + NB, t)
        return carry

    assert JPT >= 3 * NB
    k_end = (JPT - 2 * NB) // NB
    lax.fori_loop(1, k_end + 1, _steady, 0)

    for j in range(NB * (k_end + 1), JPT):
        X = j % NB
        if j + 1 < JPT:
            s2(j + 1, (j + 1) % NB, False)
        s3(j, X, j + NB >= JPT)
        if j + NB < JPT:
            s1(j + NB, X)

    plsc.subcore_barrier()

    pltpu.sync_copy(aggsh.at[pl.ds(row0, ROWS_PER_TILE)],
                    out_hbm.at[c, pl.ds(row0, ROWS_PER_TILE)])

    @pl.when(s == 15)
    def _write_tail():
        pltpu.sync_copy(aggsh.at[pl.ds(16 * ROWS_PER_TILE,
                                       N - 16 * ROWS_PER_TILE)],
                        out_hbm.at[c, pl.ds(16 * ROWS_PER_TILE,
                                            N - 16 * ROWS_PER_TILE)])


def _sc_scatter(wc, xn, senders, receivers, h):
    mesh = plsc.VectorSubcoreMesh(core_axis_name="c", subcore_axis_name="s")
    f = pl.kernel(
        functools.partial(_sc_scatter_body, h),
        out_type=jax.ShapeDtypeStruct((2, N, H), jnp.float32),
        mesh=mesh,
        compiler_params=pltpu.CompilerParams(needs_layout_passes=False),
        scratch_types=(
            [pltpu.VMEM((CH,), jnp.int32) for _ in range(NB)]
            + [pltpu.VMEM((CH,), jnp.int32) for _ in range(NB)]
            + [pltpu.VMEM((CH, H), jnp.float32) for _ in range(NB)]
            + [pltpu.VMEM((CH, H), jnp.float32) for _ in range(NB)]
            + [pltpu.VMEM_SHARED((N, H), jnp.float32)]
            + [pltpu.SemaphoreType.DMA for _ in range(3 * NB)]
        ),
    )
    return f(wc, xn, senders, receivers)


def _out_body(x_ref, pa_ref, pb_ref, w2a_ref, w2b_ref, b2_ref, o_ref):
    agg = (pa_ref[0] + pa_ref[1]) + (pb_ref[0] + pb_ref[1])
    o_ref[...] = (
        jnp.dot(x_ref[...], w2a_ref[...], preferred_element_type=jnp.float32)
        + jnp.dot(agg, w2b_ref[...], preferred_element_type=jnp.float32)
        + b2_ref[...]
    )


def _out_proj(x, pa, pb, W2, b2):
    return pl.pallas_call(
        _out_body,
        grid=(N // BN,),
        in_specs=[
            pl.BlockSpec((BN, H), lambda i: (i, 0)),
            pl.BlockSpec((2, BN, H), lambda i: (0, i, 0)),
            pl.BlockSpec((2, BN, H), lambda i: (0, i, 0)),
            pl.BlockSpec((H, H), lambda i: (0, 0)),
            pl.BlockSpec((H, H), lambda i: (0, 0)),
            pl.BlockSpec((1, H), lambda i: (0, 0)),
        ],
        out_specs=pl.BlockSpec((BN, H), lambda i: (i, 0)),
        out_shape=jax.ShapeDtypeStruct((N, H), jnp.float32),
    )(x, pa, pb, W2[:H], W2[H:], b2.reshape(1, H))


def kernel(z, x, senders, receivers, edge_weight, edge_attr, W1, b1, embed, W2, b2):
    senders = senders.astype(jnp.int32)
    receivers = receivers.astype(jnp.int32)
    ew3 = edge_weight.reshape(E // BE, BE // H, H)
    ea_t = edge_attr.T
    xn = _embed_lookup(z, embed)
    wc0 = _edge_features(ea_t, W1, b1, ew3, 0)
    pa = _sc_scatter(wc0, xn, senders, receivers, 0)
    wc1 = _edge_features(ea_t, W1, b1, ew3, 1)
    pb = _sc_scatter(wc1, xn, senders, receivers, 1)
    return _out_proj(x, pa, pb, W2, b2)


# speedup vs baseline: 6.2498x; 1.0018x over previous
import functools
import math

import jax
import jax.numpy as jnp
from jax import lax
from jax.experimental import pallas as pl
from jax.experimental.pallas import tpu as pltpu
from jax.experimental.pallas import tpu_sc as plsc

N = 10000
E = 320000
H = 128
D_EDGE = 16
MAX_SPECIES = 100
CUTOFF_UPPER = 5.0

L = 16
NW = 32
CH = 40
NH = 2
EH = E // NH
JPT = EH // (NW * CH)
ROWS_PER_TILE = 624

BE = 6400
BN = 2000


def _edge_matmul_body(eat_ref, w1_ref, b1_ref, ew_ref, out_ref):
    w = lax.dot_general(eat_ref[...], w1_ref[...],
                        (((0,), (0,)), ((), ())),
                        preferred_element_type=jnp.float32)
    w = w + b1_ref[...]
    ew = ew_ref[0]
    c = 0.5 * (jnp.cos(ew * (math.pi / CUTOFF_UPPER)) + 1.0)
    c = c * (ew < CUTOFF_UPPER).astype(jnp.float32)
    w3 = w.reshape(BE // H, H, H) * c[:, :, None]
    out_ref[...] = w3.reshape(BE, H)


def _edge_features(ea_t, W1, b1, ew3, h):
    off = h * (EH // BE)
    return pl.pallas_call(
        _edge_matmul_body,
        grid=(EH // BE,),
        in_specs=[
            pl.BlockSpec((D_EDGE, BE), lambda i: (0, i + off)),
            pl.BlockSpec((D_EDGE, H), lambda i: (0, 0)),
            pl.BlockSpec((1, H), lambda i: (0, 0)),
            pl.BlockSpec((1, BE // H, H), lambda i: (i + off, 0, 0)),
        ],
        out_specs=pl.BlockSpec((BE, H), lambda i: (i, 0)),
        out_shape=jax.ShapeDtypeStruct((EH, H), jnp.float32),
    )(ea_t, W1, b1.reshape(1, H), ew3)


def _embed_body(z_ref, emb_ref, out_ref):
    zb = z_ref[...]
    iota = lax.broadcasted_iota(jnp.int32, (BN, MAX_SPECIES), 1)
    oh = (zb == iota).astype(jnp.float32)
    out_ref[...] = jnp.dot(oh, emb_ref[...], preferred_element_type=jnp.float32)


def _embed_lookup(z, embed):
    z2 = z.astype(jnp.int32).reshape(N, 1)
    return pl.pallas_call(
        _embed_body,
        grid=(N // BN,),
        in_specs=[
            pl.BlockSpec((BN, 1), lambda i: (i, 0)),
            pl.BlockSpec((MAX_SPECIES, H), lambda i: (0, 0)),
        ],
        out_specs=pl.BlockSpec((BN, H), lambda i: (i, 0)),
        out_shape=jax.ShapeDtypeStruct((N, H), jnp.float32),
    )(z2, embed)


NB = 4


def _sc_scatter_body(h, wc_hbm, xn_hbm, snd_hbm, rcv_hbm, out_hbm,
                     *sc):
    sidx = sc[0:NB]
    ridx = sc[NB:2 * NB]
    wcv = sc[2 * NB:3 * NB]
    xsv = sc[3 * NB:4 * NB]
    aggsh = sc[4 * NB]
    sem_cp = sc[4 * NB + 1:5 * NB + 1]
    sem_g = sc[5 * NB + 1:6 * NB + 1]
    sem_sc = sc[6 * NB + 1:7 * NB + 1]

    c = lax.axis_index("c")
    s = lax.axis_index("s")
    wid = s * 2 + c
    tile_base = h * EH + wid * (JPT * CH)

    zv = jnp.zeros((L,), jnp.float32)

    def _zero_row(i, carry):
        for cc in range(H // L):
            xsv[0][i, pl.ds(cc * L, L)] = zv
        return carry

    lax.fori_loop(0, CH, _zero_row, 0)
    row0 = s * ROWS_PER_TILE
    for k in range(ROWS_PER_TILE // CH):
        pltpu.sync_copy(xsv[0], aggsh.at[pl.ds(row0 + k * CH, CH)])
    rem = ROWS_PER_TILE % CH
    if rem:
        pltpu.sync_copy(xsv[0].at[pl.ds(0, rem)],
                        aggsh.at[pl.ds(row0 + ROWS_PER_TILE - rem, rem)])

    @pl.when(s == 15)
    def _zero_tail():
        pltpu.sync_copy(xsv[0].at[pl.ds(0, N - 16 * ROWS_PER_TILE)],
                        aggsh.at[pl.ds(16 * ROWS_PER_TILE,
                                       N - 16 * ROWS_PER_TILE)])

    plsc.subcore_barrier()

    def s1(j, X):
        base = tile_base + j * CH
        wbase = base - h * EH
        pltpu.async_copy(snd_hbm.at[pl.ds(base, CH)], sidx[X], sem_cp[X])
        pltpu.async_copy(wc_hbm.at[pl.ds(wbase, CH)], wcv[X], sem_cp[X])

    def s2(j, X, first):
        if not first:
            pltpu.make_async_copy(xsv[X], aggsh.at[ridx[X]], sem_sc[X]).wait()
        pltpu.make_async_copy(snd_hbm.at[pl.ds(0, CH)], sidx[X],
                              sem_cp[X]).wait()
        pltpu.make_async_copy(wc_hbm.at[pl.ds(0, CH)], wcv[X],
                              sem_cp[X]).wait()
        base = tile_base + j * CH
        pltpu.async_copy(rcv_hbm.at[pl.ds(base, CH)], ridx[X], sem_g[X])
        pltpu.async_copy(xn_hbm.at[sidx[X]], xsv[X], sem_g[X])

    def _mul(X):
        @plsc.parallel_loop(0, CH, 1, unroll=4)
        def _mul_row(i):
            for g in range(H // L):
                sl = pl.ds(g * L, L)
                xsv[X][i, sl] = xsv[X][i, sl] * wcv[X][i, sl]

    def s3(j, X, last):
        pltpu.make_async_copy(rcv_hbm.at[pl.ds(0, CH)], ridx[X],
                              sem_g[X]).wait()
        pltpu.make_async_copy(xn_hbm.at[sidx[X]], xsv[X], sem_g[X]).wait()
        _mul(X)
        if last:
            pltpu.sync_copy(xsv[X], aggsh.at[ridx[X]], add=True)
        else:
            pltpu.async_copy(xsv[X], aggsh.at[ridx[X]], sem_sc[X], add=True)

    for t in range(NB):
        s1(t, t)
    s2(0, 0, True)
    for j in range(NB):
        s2(j + 1, (j + 1) % NB, j + 1 < NB)
        s3(j, j % NB, False)
        s1(j + NB, j % NB)

    def _steady(k, carry):
        for t in range(NB):
            j = NB * k + t
            s2(j + 1, (t + 1) % NB, False)
            s3(j, t, False)
            s1(j + NB, t)
        return carry

    assert JPT >= 3 * NB
    k_end = (JPT - 2 * NB) // NB
    lax.fori_loop(1, k_end + 1, _steady, 0)

    for j in range(NB * (k_end + 1), JPT):
        X = j % NB
        if j + 1 < JPT:
            s2(j + 1, (j + 1) % NB, False)
        s3(j, X, j + NB >= JPT)
        if j + NB < JPT:
            s1(j + NB, X)

    plsc.subcore_barrier()

    pltpu.sync_copy(aggsh.at[pl.ds(row0, ROWS_PER_TILE)],
                    out_hbm.at[c, pl.ds(row0, ROWS_PER_TILE)])

    @pl.when(s == 15)
    def _write_tail():
        pltpu.sync_copy(aggsh.at[pl.ds(16 * ROWS_PER_TILE,
                                       N - 16 * ROWS_PER_TILE)],
                        out_hbm.at[c, pl.ds(16 * ROWS_PER_TILE,
                                            N - 16 * ROWS_PER_TILE)])


def _sc_scatter(wc, xn, senders, receivers, h):
    mesh = plsc.VectorSubcoreMesh(core_axis_name="c", subcore_axis_name="s")
    f = pl.kernel(
        functools.partial(_sc_scatter_body, h),
        out_type=jax.ShapeDtypeStruct((2, N, H), jnp.float32),
        mesh=mesh,
        compiler_params=pltpu.CompilerParams(needs_layout_passes=False),
        scratch_types=(
            [pltpu.VMEM((CH,), jnp.int32) for _ in range(NB)]
            + [pltpu.VMEM((CH,), jnp.int32) for _ in range(NB)]
            + [pltpu.VMEM((CH, H), jnp.float32) for _ in range(NB)]
            + [pltpu.VMEM((CH, H), jnp.float32) for _ in range(NB)]
            + [pltpu.VMEM_SHARED((N, H), jnp.float32)]
            + [pltpu.SemaphoreType.DMA for _ in range(3 * NB)]
        ),
    )
    return f(wc, xn, senders, receivers)


def _out_body(x_ref, pa_ref, pb_ref, w2a_ref, w2b_ref, b2_ref, o_ref):
    agg = (pa_ref[0] + pa_ref[1]) + (pb_ref[0] + pb_ref[1])
    o_ref[...] = (
        jnp.dot(x_ref[...], w2a_ref[...], preferred_element_type=jnp.float32)
        + jnp.dot(agg, w2b_ref[...], preferred_element_type=jnp.float32)
        + b2_ref[...]
    )


def _out_proj(x, pa, pb, W2, b2):
    return pl.pallas_call(
        _out_body,
        grid=(N // BN,),
        in_specs=[
            pl.BlockSpec((BN, H), lambda i: (i, 0)),
            pl.BlockSpec((2, BN, H), lambda i: (0, i, 0)),
            pl.BlockSpec((2, BN, H), lambda i: (0, i, 0)),
            pl.BlockSpec((H, H), lambda i: (0, 0)),
            pl.BlockSpec((H, H), lambda i: (0, 0)),
            pl.BlockSpec((1, H), lambda i: (0, 0)),
        ],
        out_specs=pl.BlockSpec((BN, H), lambda i: (i, 0)),
        out_shape=jax.ShapeDtypeStruct((N, H), jnp.float32),
    )(x, pa, pb, W2[:H], W2[H:], b2.reshape(1, H))


def kernel(z, x, senders, receivers, edge_weight, edge_attr, W1, b1, embed, W2, b2):
    senders = senders.astype(jnp.int32)
    receivers = receivers.astype(jnp.int32)
    ew3 = edge_weight.reshape(E // BE, BE // H, H)
    ea_t = edge_attr.T
    xn = _embed_lookup(z, embed)
    wc0 = _edge_features(ea_t, W1, b1, ew3, 0)
    pa = _sc_scatter(wc0, xn, senders, receivers, 0)
    wc1 = _edge_features(ea_t, W1, b1, ew3, 1)
    pb = _sc_scatter(wc1, xn, senders, receivers, 1)
    return _out_proj(x, pa, pb, W2, b2)
